# baseline (speedup 1.0000x reference)
import functools

import jax
import jax.numpy as jnp
from jax import lax
from jax.experimental import pallas as pl
from jax.experimental.pallas import tpu as pltpu
from jax.experimental.pallas import tpu_sc as plsc

D = 1024
TOT = 4 * 8192
NC, NS = 2, 16
NW = NC * NS
PER_W = TOT // NW
C = 16
NCHUNK = PER_W // C
NBUF = 4

_mesh = plsc.VectorSubcoreMesh(
    core_axis_name="c", subcore_axis_name="s", num_cores=NC, num_subcores=NS
)


@functools.partial(
    pl.kernel,
    mesh=_mesh,
    out_type=jax.ShapeDtypeStruct((TOT, D), jnp.float32),
    scratch_types=[
        pltpu.VMEM((NCHUNK, C), jnp.int32),
        *([pltpu.VMEM((C, D), jnp.float32)] * NBUF),
        *([pltpu.SemaphoreType.DMA] * NBUF),
        *([pltpu.SemaphoreType.DMA] * NBUF),
    ],
)
def _gather_rows(idx_hbm, table_hbm, out_hbm, idx_v, *rest):
    bufs = rest[:NBUF]
    gsems = rest[NBUF : 2 * NBUF]
    wsems = rest[2 * NBUF :]

    cid = lax.axis_index("c")
    sid = lax.axis_index("s")
    wid = sid * NC + cid
    base = wid * PER_W

    pltpu.sync_copy(idx_hbm.at[wid], idx_v)

    def gather(chunk, b):
        return pltpu.make_async_copy(
            table_hbm.at[idx_v.at[chunk]], bufs[b], gsems[b]
        )

    def write(chunk, b):
        return pltpu.make_async_copy(
            bufs[b], out_hbm.at[pl.ds(base + chunk * C, C)], wsems[b]
        )

    for b in range(NBUF - 1):
        gather(b, b).start()

    def body(i, _):
        g = i * NBUF
        for b in range(NBUF):
            chunk = g + b
            bn = (b + NBUF - 1) % NBUF

            @pl.when(chunk + NBUF - 1 < NCHUNK)
            def _():
                @pl.when(chunk >= 1)
                def _():
                    write(chunk - 1, bn).wait()

                gather(chunk + NBUF - 1, bn).start()

            gather(chunk, b).wait()
            write(chunk, b).start()

        return 0

    lax.fori_loop(0, NCHUNK // NBUF, body, 0)

    for k in range(NBUF):
        chunk = NCHUNK - NBUF + k
        write(chunk, chunk % NBUF).wait()


def kernel(pos_idx, pe):
    idx = pos_idx.astype(jnp.int32).reshape(NW, NCHUNK, C)
    out = _gather_rows(idx, pe)
    return out.reshape(pos_idx.shape + (D,))
